# Initial kernel scaffold; baseline (speedup 1.0000x reference)
import jax
import jax.numpy as jnp
from jax.experimental import pallas as pl


def kernel(features, gt_boxes, im_info, conv_w, conv_b, cls_w, cls_b, box_w, box_b):
    raise NotImplementedError("write your pallas kernel here")


# speedup vs baseline: 12.0030x; 12.0030x over previous
import numpy as np
import jax
import jax.numpy as jnp
from jax.experimental import pallas as pl

_NA = 9
_PRE = 6000
_POST = 300
_THR = 0.7
_H = 64
_W = 64
_PIX = _H * _W
_TOT = _PIX * _NA
_ROWS = _TOT // 128
_MINI = np.int32(-2**31)


def _anchor_geom():
    ratios = np.array([0.5, 1.0, 2.0])
    scales = np.array([8.0, 16.0, 32.0])
    base = 16.0
    ctr = 0.5 * (base - 1.0)
    ws0 = np.round(np.sqrt(base * base / ratios))
    hs0 = np.round(ws0 * ratios)
    aw, ah = [], []
    for i in range(3):
        for s in scales:
            aw.append(ws0[i] * s)
            ah.append(hs0[i] * s)
    aw = np.array(aw, np.float32)
    ah = np.array(ah, np.float32)
    x1 = (ctr - 0.5 * (aw - 1.0)).astype(np.float32)
    y1 = (ctr - 0.5 * (ah - 1.0)).astype(np.float32)
    acx = (x1 + 0.5 * aw).astype(np.float32)
    acy = (y1 + 0.5 * ah).astype(np.float32)
    return np.stack([aw, ah, acx, acy]).astype(np.float32)


_AGEOM = _anchor_geom()


def _head_body(xpad_ref, w9_ref, cb_ref, cw_ref, cbb_ref, bw_ref, bbb_ref,
               geo_ref, ag_ref, sc_ref, x1_ref, y1_ref, x2_ref, y2_ref):
    acc = jnp.zeros((_PIX, 512), jnp.float32)
    for k in range(9):
        ky, kx = divmod(k, 3)
        patch = xpad_ref[ky:ky + _H, kx:kx + _W, :].reshape(_PIX, 256)
        acc = acc + jnp.dot(patch, w9_ref[k],
                            preferred_element_type=jnp.float32)
    rpn = jnp.maximum(acc + cb_ref[...], 0.0)

    cls = jnp.dot(rpn, cw_ref[...],
                  preferred_element_type=jnp.float32) + cbb_ref[...]
    s0 = cls[:, 0:_NA]
    s1 = cls[:, _NA:2 * _NA]
    mx = jnp.maximum(s0, s1)
    e0 = jnp.exp(s0 - mx)
    e1 = jnp.exp(s1 - mx)
    scores = e1 / (e0 + e1)

    boxd = jnp.dot(rpn, bw_ref[...],
                   preferred_element_type=jnp.float32) + bbb_ref[...]
    dx = boxd[:, 0:9]
    dy = boxd[:, 9:18]
    dw = boxd[:, 18:27]
    dh = boxd[:, 27:36]

    aw = ag_ref[0:1, :]
    ah = ag_ref[1:2, :]
    pix = jax.lax.broadcasted_iota(jnp.int32, (_PIX, _NA), 0)
    px = ((pix % _W) * 16).astype(jnp.float32)
    py = ((pix // _W) * 16).astype(jnp.float32)
    cx = ag_ref[2:3, :] + px
    cy = ag_ref[3:4, :] + py

    pcx = dx * aw + cx
    pcy = dy * ah + cy
    pw = jnp.exp(dw) * aw
    ph = jnp.exp(dh) * ah
    x1 = pcx - 0.5 * pw
    y1 = pcy - 0.5 * ph
    x2 = pcx + 0.5 * pw
    y2 = pcy + 0.5 * ph

    Hm1 = geo_ref[0:1, 0:1] - 1.0
    Wm1 = geo_ref[0:1, 1:2] - 1.0
    msz = 16.0 * geo_ref[0:1, 2:3]
    x1c = jnp.clip(x1, 0.0, Wm1)
    y1c = jnp.clip(y1, 0.0, Hm1)
    x2c = jnp.clip(x2, 0.0, Wm1)
    y2c = jnp.clip(y2, 0.0, Hm1)

    ws = x2c - x1c + 1.0
    hs = y2c - y1c + 1.0
    valid = (ws >= msz) & (hs >= msz)
    sc_ref[...] = jnp.where(valid, scores, -1e9)
    x1_ref[...] = x1c
    y1_ref[...] = y1c
    x2_ref[...] = x2c
    y2_ref[...] = y2c


def _nms_body(sc_ref, x1_ref, y1_ref, x2_ref, y2_ref, out_ref):
    s = sc_ref[...]
    x1 = x1_ref[...]
    y1 = y1_ref[...]
    x2 = x2_ref[...]
    y2 = y2_ref[...]

    bits = jax.lax.bitcast_convert_type(s, jnp.int32)
    key = bits ^ ((bits >> 31) & np.int32(0x7FFFFFFF))
    tu = jnp.int32(0)
    for b in range(31, -1, -1):
        bit = _MINI if b == 31 else np.int32(1 << b)
        cand = tu | bit
        cnt = jnp.sum((key >= (cand ^ _MINI)).astype(jnp.int32))
        tu = jnp.where(cnt >= _PRE, cand, tu)
    kt = tu ^ _MINI
    s = jnp.where(key >= kt, s, -jnp.inf)

    areas = (x2 - x1 + 1.0) * (y2 - y1 + 1.0)
    fidx = (jax.lax.broadcasted_iota(jnp.int32, (_ROWS, 128), 0) * 128
            + jax.lax.broadcasted_iota(jnp.int32, (_ROWS, 128), 1))
    riota = jax.lax.broadcasted_iota(jnp.int32, (304, 8), 0)
    liota = jax.lax.broadcasted_iota(jnp.int32, (304, 8), 1)

    def body(i, carry):
        s, outv, j0 = carry
        m = jnp.max(s)
        j = jnp.min(jnp.where(s == m, fidx, jnp.int32(1 << 30)))
        j = jnp.where(m == -jnp.inf, j0, j)
        j0 = jnp.where(i == 0, j, j0)
        sel = fidx == j
        bx1 = jnp.sum(jnp.where(sel, x1, 0.0))
        by1 = jnp.sum(jnp.where(sel, y1, 0.0))
        bx2 = jnp.sum(jnp.where(sel, x2, 0.0))
        by2 = jnp.sum(jnp.where(sel, y2, 0.0))
        bar = jnp.sum(jnp.where(sel, areas, 0.0))
        xx1 = jnp.maximum(bx1, x1)
        yy1 = jnp.maximum(by1, y1)
        xx2 = jnp.minimum(bx2, x2)
        yy2 = jnp.minimum(by2, y2)
        iw = jnp.maximum(0.0, xx2 - xx1 + 1.0)
        ih = jnp.maximum(0.0, yy2 - yy1 + 1.0)
        inter = iw * ih
        ovr = inter / (bar + areas - inter)
        s = jnp.where(ovr > _THR, -jnp.inf, s)
        nr = jnp.where(liota == 1, bx1,
                       jnp.where(liota == 2, by1,
                                 jnp.where(liota == 3, bx2,
                                           jnp.where(liota == 4, by2, 0.0))))
        outv = jnp.where(riota == i, nr, outv)
        return s, outv, j0

    _, outv, _ = jax.lax.fori_loop(
        0, _POST, body, (s, jnp.zeros((304, 8), jnp.float32), jnp.int32(0)))
    out_ref[...] = outv


def _run_head(xpad, w9, cb, cw, cbb, bw, bbb, geo, interpret=False):
    shp = jax.ShapeDtypeStruct((_PIX, _NA), jnp.float32)
    return pl.pallas_call(
        _head_body,
        out_shape=[shp] * 5,
        interpret=interpret,
    )(xpad, w9, cb, cw, cbb, bw, bbb, geo, jnp.asarray(_AGEOM))


def _run_nms(sc, x1, y1, x2, y2, interpret=False):
    return pl.pallas_call(
        _nms_body,
        out_shape=jax.ShapeDtypeStruct((304, 8), jnp.float32),
        interpret=interpret,
    )(sc, x1, y1, x2, y2)


def _kernel_impl(features, gt_boxes, im_info, conv_w, conv_b, cls_w, cls_b,
                 box_w, box_b, interpret=False):
    x = features[0].transpose(1, 2, 0)
    xpad = jnp.pad(x, ((1, 1), (1, 1), (0, 0)))
    w9 = conv_w.transpose(2, 3, 1, 0).reshape(9, 256, 512)
    cb = conv_b.reshape(1, 512)
    cw = cls_w[:, :, 0, 0].T
    cbb = cls_b.reshape(1, 18)
    perm = np.array([a * 4 + d for d in range(4) for a in range(_NA)])
    bw = box_w[:, :, 0, 0].T[:, perm]
    bbb = box_b[perm].reshape(1, 36)
    geo = jnp.pad(im_info, ((0, 0), (0, 125)))

    sc, x1, y1, x2, y2 = _run_head(xpad, w9, cb, cw, cbb, bw, bbb, geo,
                                   interpret=interpret)

    def _r(t):
        return t.reshape(_TOT).reshape(_ROWS, 128)

    out = _run_nms(_r(sc), _r(x1), _r(y1), _r(x2), _r(y2),
                   interpret=interpret)
    return out[:_POST, :5]


def kernel(features, gt_boxes, im_info, conv_w, conv_b, cls_w, cls_b,
           box_w, box_b):
    return _kernel_impl(features, gt_boxes, im_info, conv_w, conv_b,
                        cls_w, cls_b, box_w, box_b)


# speedup vs baseline: 13.6141x; 1.1342x over previous
import numpy as np
import jax
import jax.numpy as jnp
from jax.experimental import pallas as pl

_NA = 9
_PRE = 6000
_POST = 300
_THR = 0.7
_H = 64
_W = 64
_PIX = _H * _W
_TOT = _PIX * _NA
_ROWS = _TOT // 128
_MINI = np.int32(-2**31)


def _anchor_geom():
    ratios = np.array([0.5, 1.0, 2.0])
    scales = np.array([8.0, 16.0, 32.0])
    base = 16.0
    ctr = 0.5 * (base - 1.0)
    ws0 = np.round(np.sqrt(base * base / ratios))
    hs0 = np.round(ws0 * ratios)
    aw, ah = [], []
    for i in range(3):
        for s in scales:
            aw.append(ws0[i] * s)
            ah.append(hs0[i] * s)
    aw = np.array(aw, np.float32)
    ah = np.array(ah, np.float32)
    x1 = (ctr - 0.5 * (aw - 1.0)).astype(np.float32)
    y1 = (ctr - 0.5 * (ah - 1.0)).astype(np.float32)
    acx = (x1 + 0.5 * aw).astype(np.float32)
    acy = (y1 + 0.5 * ah).astype(np.float32)
    return np.stack([aw, ah, acx, acy]).astype(np.float32)


_AGEOM = _anchor_geom()


def _head_body(xpad_ref, w9_ref, cb_ref, cw_ref, cbb_ref, bw_ref, bbb_ref,
               geo_ref, ag_ref, sc_ref, x1_ref, y1_ref, x2_ref, y2_ref):
    acc = jnp.zeros((_PIX, 512), jnp.float32)
    for k in range(9):
        ky, kx = divmod(k, 3)
        patch = xpad_ref[ky:ky + _H, kx:kx + _W, :].reshape(_PIX, 256)
        acc = acc + jnp.dot(patch, w9_ref[k],
                            preferred_element_type=jnp.float32)
    rpn = jnp.maximum(acc + cb_ref[...], 0.0)

    cls = jnp.dot(rpn, cw_ref[...],
                  preferred_element_type=jnp.float32) + cbb_ref[...]
    s0 = cls[:, 0:_NA]
    s1 = cls[:, _NA:2 * _NA]
    mx = jnp.maximum(s0, s1)
    e0 = jnp.exp(s0 - mx)
    e1 = jnp.exp(s1 - mx)
    scores = e1 / (e0 + e1)

    boxd = jnp.dot(rpn, bw_ref[...],
                   preferred_element_type=jnp.float32) + bbb_ref[...]
    dx = boxd[:, 0:9]
    dy = boxd[:, 9:18]
    dw = boxd[:, 18:27]
    dh = boxd[:, 27:36]

    aw = ag_ref[0:1, :]
    ah = ag_ref[1:2, :]
    pix = jax.lax.broadcasted_iota(jnp.int32, (_PIX, _NA), 0)
    px = ((pix % _W) * 16).astype(jnp.float32)
    py = ((pix // _W) * 16).astype(jnp.float32)
    cx = ag_ref[2:3, :] + px
    cy = ag_ref[3:4, :] + py

    pcx = dx * aw + cx
    pcy = dy * ah + cy
    pw = jnp.exp(dw) * aw
    ph = jnp.exp(dh) * ah
    x1 = pcx - 0.5 * pw
    y1 = pcy - 0.5 * ph
    x2 = pcx + 0.5 * pw
    y2 = pcy + 0.5 * ph

    Hm1 = geo_ref[0:1, 0:1] - 1.0
    Wm1 = geo_ref[0:1, 1:2] - 1.0
    msz = 16.0 * geo_ref[0:1, 2:3]
    x1c = jnp.clip(x1, 0.0, Wm1)
    y1c = jnp.clip(y1, 0.0, Hm1)
    x2c = jnp.clip(x2, 0.0, Wm1)
    y2c = jnp.clip(y2, 0.0, Hm1)

    ws = x2c - x1c + 1.0
    hs = y2c - y1c + 1.0
    valid = (ws >= msz) & (hs >= msz)
    sc_ref[...] = jnp.where(valid, scores, -1e9)
    x1_ref[...] = x1c
    y1_ref[...] = y1c
    x2_ref[...] = x2c
    y2_ref[...] = y2c


def _nms_body(sc_ref, x1_ref, y1_ref, x2_ref, y2_ref, out_ref):
    s = sc_ref[...]
    x1 = x1_ref[...]
    y1 = y1_ref[...]
    x2 = x2_ref[...]
    y2 = y2_ref[...]

    bits = jax.lax.bitcast_convert_type(s, jnp.int32)
    key = bits ^ ((bits >> 31) & np.int32(0x7FFFFFFF))
    tu = jnp.int32(0)
    for b in range(31, -1, -1):
        bit = _MINI if b == 31 else np.int32(1 << b)
        cand = tu | bit
        cnt = jnp.sum((key >= (cand ^ _MINI)).astype(jnp.int32))
        tu = jnp.where(cnt >= _PRE, cand, tu)
    kt = tu ^ _MINI
    s = jnp.where(key >= kt, s, -jnp.inf)

    areas = (x2 - x1 + 1.0) * (y2 - y1 + 1.0)
    xp2 = x2 + 1.0
    yp2 = y2 + 1.0
    fidx = (jax.lax.broadcasted_iota(jnp.int32, (_ROWS, 128), 0) * 128
            + jax.lax.broadcasted_iota(jnp.int32, (_ROWS, 128), 1))
    l128 = jax.lax.broadcasted_iota(jnp.int32, (1, 128), 1)
    l8 = jax.lax.broadcasted_iota(jnp.int32, (1, 8), 1)

    def body(i, carry):
        s, j0 = carry
        m = jnp.max(s)
        j = jnp.min(jnp.where(s == m, fidx, jnp.int32(1 << 30)))
        j = jnp.where(m == -jnp.inf, j0, j)
        j0 = jnp.where(i == 0, j, j0)
        row = j // 128
        lane = j % 128
        lsel = l128 == lane

        def ext(ref):
            return jnp.sum(jnp.where(lsel, ref[pl.ds(row, 1), :], 0.0))

        bx1 = ext(x1_ref)
        by1 = ext(y1_ref)
        bx2 = ext(x2_ref)
        by2 = ext(y2_ref)
        bar = (bx2 - bx1 + 1.0) * (by2 - by1 + 1.0)
        iw = jnp.maximum(0.0, jnp.minimum(bx2 + 1.0, xp2)
                         - jnp.maximum(bx1, x1))
        ih = jnp.maximum(0.0, jnp.minimum(by2 + 1.0, yp2)
                         - jnp.maximum(by1, y1))
        inter = iw * ih
        den = (areas + bar) - inter
        s = jnp.where(inter > _THR * den, -jnp.inf, s)
        nr = jnp.where(l8 == 1, bx1,
                       jnp.where(l8 == 2, by1,
                                 jnp.where(l8 == 3, bx2,
                                           jnp.where(l8 == 4, by2, 0.0))))
        out_ref[pl.ds(i, 1), :] = nr
        return s, j0

    jax.lax.fori_loop(0, _POST, body, (s, jnp.int32(0)))


def _run_head(xpad, w9, cb, cw, cbb, bw, bbb, geo, interpret=False):
    shp = jax.ShapeDtypeStruct((_PIX, _NA), jnp.float32)
    return pl.pallas_call(
        _head_body,
        out_shape=[shp] * 5,
        interpret=interpret,
    )(xpad, w9, cb, cw, cbb, bw, bbb, geo, jnp.asarray(_AGEOM))


def _run_nms(sc, x1, y1, x2, y2, interpret=False):
    return pl.pallas_call(
        _nms_body,
        out_shape=jax.ShapeDtypeStruct((304, 8), jnp.float32),
        interpret=interpret,
    )(sc, x1, y1, x2, y2)


def _kernel_impl(features, gt_boxes, im_info, conv_w, conv_b, cls_w, cls_b,
                 box_w, box_b, interpret=False):
    x = features[0].transpose(1, 2, 0)
    xpad = jnp.pad(x, ((1, 1), (1, 1), (0, 0)))
    w9 = conv_w.transpose(2, 3, 1, 0).reshape(9, 256, 512)
    cb = conv_b.reshape(1, 512)
    cw = cls_w[:, :, 0, 0].T
    cbb = cls_b.reshape(1, 18)
    perm = np.array([a * 4 + d for d in range(4) for a in range(_NA)])
    bw = box_w[:, :, 0, 0].T[:, perm]
    bbb = box_b[perm].reshape(1, 36)
    geo = jnp.pad(im_info, ((0, 0), (0, 125)))

    sc, x1, y1, x2, y2 = _run_head(xpad, w9, cb, cw, cbb, bw, bbb, geo,
                                   interpret=interpret)

    def _r(t):
        return t.reshape(_TOT).reshape(_ROWS, 128)

    out = _run_nms(_r(sc), _r(x1), _r(y1), _r(x2), _r(y2),
                   interpret=interpret)
    return out[:_POST, :5]


def kernel(features, gt_boxes, im_info, conv_w, conv_b, cls_w, cls_b,
           box_w, box_b):
    return _kernel_impl(features, gt_boxes, im_info, conv_w, conv_b,
                        cls_w, cls_b, box_w, box_b)
